# baseline (speedup 1.0000x reference)
import functools

import jax
import jax.numpy as jnp
from jax import lax
from jax.experimental import pallas as pl
from jax.experimental.pallas import tpu as pltpu
from jax.experimental.pallas import tpu_sc as plsc

VOCAB = 1000000
EMBED_DIM = 64
BATCH = 4096
HIST = 200
N = BATCH * HIST

_info = plsc.get_sparse_core_info()
NC = _info.num_cores
NS = _info.num_subcores
NW = NC * NS
PER_W = N // NW

IDX_W = 128
GROUPS = 8
CHUNK = IDX_W * GROUPS
STEPS = PER_W // CHUNK
ROWS_PER_W = PER_W // IDX_W

_mesh = plsc.VectorSubcoreMesh(core_axis_name="c", subcore_axis_name="s")


@functools.partial(
    pl.kernel,
    mesh=_mesh,
    out_type=jax.ShapeDtypeStruct((N, EMBED_DIM), jnp.float32),
    scratch_types=[
        pltpu.VMEM((GROUPS, IDX_W), jnp.int32),
        pltpu.VMEM((CHUNK, EMBED_DIM), jnp.float32),
        pltpu.SemaphoreType.DMA,
    ],
    compiler_params=pltpu.CompilerParams(use_tc_tiling_on_sc=False),
)
def _gather_kernel(idx_hbm, table_hbm, out_hbm, idx_v, rows_v, sem):
    wid = lax.axis_index("s") * NC + lax.axis_index("c")
    row_base = wid * ROWS_PER_W
    out_base = wid * PER_W

    def step(i, carry):
        pltpu.sync_copy(idx_hbm.at[pl.ds(row_base + i * GROUPS, GROUPS), :], idx_v)
        for g in range(GROUPS):
            pltpu.async_copy(
                table_hbm.at[idx_v.at[g]],
                rows_v.at[pl.ds(g * IDX_W, IDX_W), :],
                sem,
            )
        for g in range(GROUPS):
            pltpu.make_async_copy(
                table_hbm.at[idx_v.at[g]],
                rows_v.at[pl.ds(g * IDX_W, IDX_W), :],
                sem,
            ).wait()
        pltpu.sync_copy(rows_v, out_hbm.at[pl.ds(out_base + i * CHUNK, CHUNK), :])
        return carry

    lax.fori_loop(0, STEPS, step, 0)


def kernel(x, table):
    flat = x.reshape(N // IDX_W, IDX_W).astype(jnp.int32)
    out = _gather_kernel(flat, table)
    return out.reshape(BATCH, HIST, EMBED_DIM)


# speedup vs baseline: 1.0189x; 1.0189x over previous
import functools

import jax
import jax.numpy as jnp
from jax import lax
from jax.experimental import pallas as pl
from jax.experimental.pallas import tpu as pltpu
from jax.experimental.pallas import tpu_sc as plsc

VOCAB = 1000000
EMBED_DIM = 64
BATCH = 4096
HIST = 200
N = BATCH * HIST

_info = plsc.get_sparse_core_info()
NC = _info.num_cores
NS = _info.num_subcores
NW = NC * NS
PER_W = N // NW

IDX_W = 128
GROUPS = 4
CHUNK = IDX_W * GROUPS
STEPS = PER_W // CHUNK
PAIRS = STEPS // 2
ROWS_PER_W = PER_W // IDX_W

_mesh = plsc.VectorSubcoreMesh(core_axis_name="c", subcore_axis_name="s")


@functools.partial(
    pl.kernel,
    mesh=_mesh,
    out_type=jax.ShapeDtypeStruct((N, EMBED_DIM), jnp.float32),
    scratch_types=[
        pltpu.VMEM((ROWS_PER_W, IDX_W), jnp.int32),
        pltpu.VMEM((CHUNK, EMBED_DIM), jnp.float32),
        pltpu.VMEM((CHUNK, EMBED_DIM), jnp.float32),
        pltpu.SemaphoreType.DMA,
        pltpu.SemaphoreType.DMA,
    ],
    compiler_params=pltpu.CompilerParams(use_tc_tiling_on_sc=False),
)
def _gather_kernel(idx_hbm, table_hbm, out_hbm, idx_v, rows0, rows1, sem0, sem1):
    wid = lax.axis_index("s") * NC + lax.axis_index("c")
    out_base = wid * PER_W

    pltpu.sync_copy(idx_hbm.at[pl.ds(wid * ROWS_PER_W, ROWS_PER_W), :], idx_v)

    rows = (rows0, rows1)
    sems = (sem0, sem1)

    def fire(c, b):
        for g in range(GROUPS):
            pltpu.async_copy(
                table_hbm.at[idx_v.at[c * GROUPS + g]],
                rows[b].at[pl.ds(g * IDX_W, IDX_W), :],
                sems[b],
            )

    def drain(c, b):
        for g in range(GROUPS):
            pltpu.make_async_copy(
                table_hbm.at[idx_v.at[c * GROUPS + g]],
                rows[b].at[pl.ds(g * IDX_W, IDX_W), :],
                sems[b],
            ).wait()

    fire(0, 0)

    def pair(j, carry):
        c0 = 2 * j
        fire(c0 + 1, 1)
        drain(c0, 0)
        pltpu.sync_copy(rows0, out_hbm.at[pl.ds(out_base + c0 * CHUNK, CHUNK), :])
        @pl.when(j < PAIRS - 1)
        def _():
            fire(c0 + 2, 0)
        drain(c0 + 1, 1)
        pltpu.sync_copy(
            rows1, out_hbm.at[pl.ds(out_base + (c0 + 1) * CHUNK, CHUNK), :]
        )
        return carry

    lax.fori_loop(0, PAIRS, pair, 0)


def kernel(x, table):
    flat = x.reshape(N // IDX_W, IDX_W).astype(jnp.int32)
    out = _gather_kernel(flat, table)
    return out.reshape(BATCH, HIST, EMBED_DIM)
